# Initial kernel scaffold; baseline (speedup 1.0000x reference)
import jax
import jax.numpy as jnp
from jax.experimental import pallas as pl


def kernel(x, edge_weight, W_rel1, b_rel1, W_root1, W_rel2, b_rel2, W_root2, W_lin, b_lin, edge_index):
    raise NotImplementedError("write your pallas kernel here")


# speedup vs baseline: 9.0123x; 9.0123x over previous
import functools

import jax
import jax.numpy as jnp
from jax import lax
from jax.experimental import pallas as pl
from jax.experimental.pallas import tpu as pltpu
from jax.experimental.pallas import tpu_sc as plsc

NB = 1000


def _mm2_body(x_ref, wr_ref, wo_ref, t_ref, r_ref):
    xb = x_ref[...]
    t_ref[...] = lax.dot_general(xb, wr_ref[...], (((1,), (1,)), ((), ())),
                                 preferred_element_type=jnp.float32)
    r_ref[...] = lax.dot_general(xb, wo_ref[...], (((1,), (1,)), ((), ())),
                                 preferred_element_type=jnp.float32)


def _mm2(x, w_rel, w_root):
    n, f = x.shape
    h = w_rel.shape[0]
    grid = n // NB
    return pl.pallas_call(
        _mm2_body,
        grid=(grid,),
        in_specs=[
            pl.BlockSpec((NB, f), lambda i: (i, 0)),
            pl.BlockSpec((h, f), lambda i: (0, 0)),
            pl.BlockSpec((h, f), lambda i: (0, 0)),
        ],
        out_specs=[
            pl.BlockSpec((NB, h), lambda i: (i, 0)),
            pl.BlockSpec((NB, h), lambda i: (i, 0)),
        ],
        out_shape=[
            jax.ShapeDtypeStruct((n, h), jnp.float32),
            jax.ShapeDtypeStruct((n, h), jnp.float32),
        ],
    )(x, w_rel, w_root)


def _segsum_sc(t, src, dst, ew, zeros_nh):
    n, h = t.shape
    e = src.shape[0]
    nw = 32
    ew_per = e // nw
    k = 80
    nch = ew_per // k
    rows_per_tile = n // 16

    src2 = src.reshape(e // k, k)
    dst2 = dst.reshape(e // k, k)
    ew2 = ew.reshape(e // k, k)

    mesh = plsc.VectorSubcoreMesh(core_axis_name="c", subcore_axis_name="s")

    @functools.partial(
        pl.kernel,
        out_type=jax.ShapeDtypeStruct((2 * n, h), jnp.float32),
        mesh=mesh,
        compiler_params=pltpu.CompilerParams(use_tc_tiling_on_sc=False),
        scratch_types=[
            pltpu.VMEM((nch, k), jnp.int32),
            pltpu.VMEM((nch, k), jnp.int32),
            pltpu.VMEM((nch, k), jnp.float32),
            pltpu.VMEM((k, h), jnp.float32),
            pltpu.VMEM_SHARED((n, h), jnp.float32),
            pltpu.SemaphoreType.DMA,
        ],
    )
    def seg_kernel(t_hbm, src_hbm, dst_hbm, ew_hbm, z_hbm, out_hbm,
                   src_v, dst_v, ew_v, rows_v, acc_sh, sem):
        c = lax.axis_index("c")
        s = lax.axis_index("s")
        wid = c * 16 + s
        base = wid * nch

        pltpu.sync_copy(src_hbm.at[pl.ds(base, nch)], src_v)
        pltpu.sync_copy(dst_hbm.at[pl.ds(base, nch)], dst_v)
        pltpu.sync_copy(ew_hbm.at[pl.ds(base, nch)], ew_v)

        pltpu.sync_copy(z_hbm.at[pl.ds(s * rows_per_tile, rows_per_tile)],
                        acc_sh.at[pl.ds(s * rows_per_tile, rows_per_tile)])
        plsc.subcore_barrier()

        def chunk_body(j, carry):
            pltpu.async_copy(t_hbm.at[src_v.at[j]], rows_v, sem).wait()

            def group_body(g, carry2):
                wv = ew_v[j, pl.ds(g * 16, 16)]
                for l in range(16):
                    r = g * 16 + l
                    wb = jnp.full((16,), wv[l], jnp.float32)
                    rows_v[r, pl.ds(0, 16)] = rows_v[r, pl.ds(0, 16)] * wb
                    rows_v[r, pl.ds(16, 16)] = rows_v[r, pl.ds(16, 16)] * wb
                return carry2

            lax.fori_loop(0, k // 16, group_body, 0, unroll=False)
            pltpu.sync_copy(rows_v, acc_sh.at[dst_v.at[j]], add=True)
            return carry

        lax.fori_loop(0, nch, chunk_body, 0, unroll=False)
        plsc.subcore_barrier()

        pltpu.sync_copy(
            acc_sh.at[pl.ds(s * rows_per_tile, rows_per_tile)],
            out_hbm.at[pl.ds(c * n + s * rows_per_tile, rows_per_tile)])

    return seg_kernel(t, src2, dst2, ew2, zeros_nh)


def _mid_body(agg_ref, r_ref, b_ref, wr2_ref, wo2_ref, x1_ref, t2_ref, r2_ref):
    i = pl.program_id(0)
    n = r_ref.shape[0] * pl.num_programs(0)
    a = agg_ref[pl.ds(i * NB, NB), :] + agg_ref[pl.ds(n + i * NB, NB), :]
    x1 = jnp.maximum(a + b_ref[...] + r_ref[...], 0.0)
    x1_ref[...] = x1
    t2_ref[...] = lax.dot_general(x1, wr2_ref[...], (((1,), (1,)), ((), ())),
                                  preferred_element_type=jnp.float32)
    r2_ref[...] = lax.dot_general(x1, wo2_ref[...], (((1,), (1,)), ((), ())),
                                  preferred_element_type=jnp.float32)


def _mid(agg2n, r1, b1, w_rel2, w_root2):
    n, h = r1.shape
    grid = n // NB
    return pl.pallas_call(
        _mid_body,
        grid=(grid,),
        in_specs=[
            pl.BlockSpec((2 * n, h), lambda i: (0, 0)),
            pl.BlockSpec((NB, h), lambda i: (i, 0)),
            pl.BlockSpec((1, h), lambda i: (0, 0)),
            pl.BlockSpec((h, h), lambda i: (0, 0)),
            pl.BlockSpec((h, h), lambda i: (0, 0)),
        ],
        out_specs=[
            pl.BlockSpec((NB, h), lambda i: (i, 0)),
            pl.BlockSpec((NB, h), lambda i: (i, 0)),
            pl.BlockSpec((NB, h), lambda i: (i, 0)),
        ],
        out_shape=[
            jax.ShapeDtypeStruct((n, h), jnp.float32),
            jax.ShapeDtypeStruct((n, h), jnp.float32),
            jax.ShapeDtypeStruct((n, h), jnp.float32),
        ],
    )(agg2n, r1, b1, w_rel2, w_root2)


def _head_body(agg_ref, r_ref, b_ref, x1_ref, wl_ref, bl_ref, out_ref):
    i = pl.program_id(0)
    n = r_ref.shape[0] * pl.num_programs(0)
    a = agg_ref[pl.ds(i * NB, NB), :] + agg_ref[pl.ds(n + i * NB, NB), :]
    x2 = jnp.maximum(a + b_ref[...] + r_ref[...], 0.0)
    hcat = jnp.concatenate([x1_ref[...], x2], axis=1)
    o = lax.dot_general(hcat, wl_ref[...], (((1,), (1,)), ((), ())),
                        preferred_element_type=jnp.float32) + bl_ref[...]
    m = jnp.max(o, axis=1, keepdims=True)
    z = o - m
    lse = jnp.log(jnp.sum(jnp.exp(z), axis=1, keepdims=True))
    out_ref[...] = z - lse


def _head(agg2n, r2, b2, x1, w_lin, b_lin):
    n, h = r2.shape
    cdim = w_lin.shape[0]
    grid = n // NB
    return pl.pallas_call(
        _head_body,
        grid=(grid,),
        in_specs=[
            pl.BlockSpec((2 * n, h), lambda i: (0, 0)),
            pl.BlockSpec((NB, h), lambda i: (i, 0)),
            pl.BlockSpec((1, h), lambda i: (0, 0)),
            pl.BlockSpec((NB, h), lambda i: (i, 0)),
            pl.BlockSpec((cdim, 2 * h), lambda i: (0, 0)),
            pl.BlockSpec((1, cdim), lambda i: (0, 0)),
        ],
        out_specs=pl.BlockSpec((NB, cdim), lambda i: (i, 0)),
        out_shape=jax.ShapeDtypeStruct((n, cdim), jnp.float32),
    )(agg2n, r2, b2, x1, w_lin, b_lin)


def kernel(x, edge_weight, W_rel1, b_rel1, W_root1, W_rel2, b_rel2, W_root2,
           W_lin, b_lin, edge_index):
    n = x.shape[0]
    h = W_rel1.shape[0]
    src = edge_index[0]
    dst = edge_index[1]
    zeros_nh = jnp.zeros((n, h), jnp.float32)

    t1, r1 = _mm2(x, W_rel1, W_root1)
    agg1 = _segsum_sc(t1, src, dst, edge_weight, zeros_nh)
    x1, t2, r2 = _mid(agg1, r1, b_rel1.reshape(1, h), W_rel2, W_root2)
    agg2 = _segsum_sc(t2, src, dst, edge_weight, zeros_nh)
    return _head(agg2, r2, b_rel2.reshape(1, h), x1, W_lin,
                 b_lin.reshape(1, -1))


# speedup vs baseline: 17.5447x; 1.9467x over previous
import functools

import jax
import jax.numpy as jnp
from jax import lax
from jax.experimental import pallas as pl
from jax.experimental.pallas import tpu as pltpu
from jax.experimental.pallas import tpu_sc as plsc


def _mm2_body(x_ref, wr_ref, wo_ref, t_ref, r_ref):
    xb = x_ref[...]
    t_ref[...] = lax.dot_general(xb, wr_ref[...], (((1,), (1,)), ((), ())),
                                 preferred_element_type=jnp.float32)
    r_ref[...] = lax.dot_general(xb, wo_ref[...], (((1,), (1,)), ((), ())),
                                 preferred_element_type=jnp.float32)


def _mm2(x, w_rel, w_root):
    n, f = x.shape
    h = w_rel.shape[0]
    return pl.pallas_call(
        _mm2_body,
        out_shape=[
            jax.ShapeDtypeStruct((n, h), jnp.float32),
            jax.ShapeDtypeStruct((n, h), jnp.float32),
        ],
    )(x, w_rel, w_root)


def _segsum_sc(t, src2, dst2, ew2, zeros_nh):
    n, h = t.shape
    nch_all, k = src2.shape
    e = nch_all * k
    nw = 32
    nch = (e // nw) // k
    sup = 5
    nsup = nch // sup
    npairs = (nsup - 1) // 2
    bk = sup * k
    rows_per_tile = n // 16

    mesh = plsc.VectorSubcoreMesh(core_axis_name="c", subcore_axis_name="s")

    @functools.partial(
        pl.kernel,
        out_type=jax.ShapeDtypeStruct((2 * n, h), jnp.float32),
        mesh=mesh,
        compiler_params=pltpu.CompilerParams(use_tc_tiling_on_sc=False),
        scratch_types=[
            pltpu.VMEM((nch, k), jnp.int32),
            pltpu.VMEM((nch, k), jnp.int32),
            pltpu.VMEM((nch, k), jnp.float32),
            pltpu.VMEM((bk, h), jnp.float32),
            pltpu.VMEM((bk, h), jnp.float32),
            pltpu.VMEM_SHARED((n, h), jnp.float32),
            pltpu.SemaphoreType.DMA,
            pltpu.SemaphoreType.DMA,
        ],
    )
    def seg_kernel(t_hbm, src_hbm, dst_hbm, ew_hbm, z_hbm, out_hbm,
                   src_v, dst_v, ew_v, buf_a, buf_b, acc_sh, sem_a, sem_b):
        c = lax.axis_index("c")
        s = lax.axis_index("s")
        wid = c * 16 + s
        base = wid * nch

        pltpu.sync_copy(src_hbm.at[pl.ds(base, nch)], src_v)
        pltpu.sync_copy(dst_hbm.at[pl.ds(base, nch)], dst_v)
        pltpu.sync_copy(ew_hbm.at[pl.ds(base, nch)], ew_v)

        pltpu.sync_copy(z_hbm.at[pl.ds(s * rows_per_tile, rows_per_tile)],
                        acc_sh.at[pl.ds(s * rows_per_tile, rows_per_tile)])
        plsc.subcore_barrier()

        def fire(sidx, buf, sem):
            for cc in range(sup):
                pltpu.async_copy(
                    t_hbm.at[src_v.at[sidx * sup + cc]],
                    buf.at[pl.ds(cc * k, k)], sem)

        def drain(sidx, buf, sem):
            for cc in range(sup):
                pltpu.make_async_copy(
                    t_hbm.at[src_v.at[sidx * sup + cc]],
                    buf.at[pl.ds(cc * k, k)], sem).wait()

        def mult(sidx, buf):
            def group_body(g, carry):
                cc = g // 5
                wv = ew_v[sidx * sup + cc, pl.ds((g - cc * 5) * 16, 16)]
                for l in range(16):
                    r = g * 16 + l
                    wb = jnp.full((16,), wv[l], jnp.float32)
                    buf[r, pl.ds(0, 16)] = buf[r, pl.ds(0, 16)] * wb
                    buf[r, pl.ds(16, 16)] = buf[r, pl.ds(16, 16)] * wb
                return carry

            lax.fori_loop(0, bk // 16, group_body, 0, unroll=False)

        def scatter(sidx, buf):
            for cc in range(sup):
                pltpu.sync_copy(buf.at[pl.ds(cc * k, k)],
                                acc_sh.at[dst_v.at[sidx * sup + cc]],
                                add=True)

        def process(sidx, buf, sem):
            drain(sidx, buf, sem)
            mult(sidx, buf)
            scatter(sidx, buf)

        fire(0, buf_a, sem_a)
        fire(1, buf_b, sem_b)
        process(0, buf_a, sem_a)

        def pair_body(p, carry):
            s1 = 2 * p + 1
            fire(s1 + 1, buf_a, sem_a)
            process(s1, buf_b, sem_b)
            s2 = 2 * p + 2

            @pl.when(s2 + 1 < nsup)
            def _():
                fire(s2 + 1, buf_b, sem_b)

            process(s2, buf_a, sem_a)
            return carry

        lax.fori_loop(0, npairs, pair_body, 0, unroll=False)
        plsc.subcore_barrier()

        pltpu.sync_copy(
            acc_sh.at[pl.ds(s * rows_per_tile, rows_per_tile)],
            out_hbm.at[pl.ds(c * n + s * rows_per_tile, rows_per_tile)])

    return seg_kernel(t, src2, dst2, ew2, zeros_nh)


def _mid_body(agg_ref, r_ref, b_ref, wr2_ref, wo2_ref, x1_ref, t2_ref, r2_ref):
    n = r_ref.shape[0]
    a = agg_ref[pl.ds(0, n), :] + agg_ref[pl.ds(n, n), :]
    x1 = jnp.maximum(a + b_ref[...] + r_ref[...], 0.0)
    x1_ref[...] = x1
    t2_ref[...] = lax.dot_general(x1, wr2_ref[...], (((1,), (1,)), ((), ())),
                                  preferred_element_type=jnp.float32)
    r2_ref[...] = lax.dot_general(x1, wo2_ref[...], (((1,), (1,)), ((), ())),
                                  preferred_element_type=jnp.float32)


def _mid(agg2n, r1, b1, w_rel2, w_root2):
    n, h = r1.shape
    return pl.pallas_call(
        _mid_body,
        out_shape=[
            jax.ShapeDtypeStruct((n, h), jnp.float32),
            jax.ShapeDtypeStruct((n, h), jnp.float32),
            jax.ShapeDtypeStruct((n, h), jnp.float32),
        ],
    )(agg2n, r1, b1, w_rel2, w_root2)


def _head_body(agg_ref, r_ref, b_ref, x1_ref, wl_ref, bl_ref, out_ref):
    n = r_ref.shape[0]
    a = agg_ref[pl.ds(0, n), :] + agg_ref[pl.ds(n, n), :]
    x2 = jnp.maximum(a + b_ref[...] + r_ref[...], 0.0)
    hcat = jnp.concatenate([x1_ref[...], x2], axis=1)
    o = lax.dot_general(hcat, wl_ref[...], (((1,), (1,)), ((), ())),
                        preferred_element_type=jnp.float32) + bl_ref[...]
    m = jnp.max(o, axis=1, keepdims=True)
    z = o - m
    lse = jnp.log(jnp.sum(jnp.exp(z), axis=1, keepdims=True))
    out_ref[...] = z - lse


def _head(agg2n, r2, b2, x1, w_lin, b_lin):
    n, h = r2.shape
    cdim = w_lin.shape[0]
    return pl.pallas_call(
        _head_body,
        out_shape=jax.ShapeDtypeStruct((n, cdim), jnp.float32),
    )(agg2n, r2, b2, x1, w_lin, b_lin)


def kernel(x, edge_weight, W_rel1, b_rel1, W_root1, W_rel2, b_rel2, W_root2,
           W_lin, b_lin, edge_index):
    n = x.shape[0]
    h = W_rel1.shape[0]
    e = edge_weight.shape[0]
    k = 80
    src2 = edge_index[0].reshape(e // k, k)
    dst2 = edge_index[1].reshape(e // k, k)
    ew2 = edge_weight.reshape(e // k, k)
    zeros_nh = jnp.zeros((n, h), jnp.float32)

    t1, r1 = _mm2(x, W_rel1, W_root1)
    agg1 = _segsum_sc(t1, src2, dst2, ew2, zeros_nh)
    x1, t2, r2 = _mid(agg1, r1, b_rel1.reshape(1, h), W_rel2, W_root2)
    agg2 = _segsum_sc(t2, src2, dst2, ew2, zeros_nh)
    return _head(agg2, r2, b_rel2.reshape(1, h), x1, W_lin,
                 b_lin.reshape(1, -1))
